# baseline (speedup 1.0000x reference)
import functools

import numpy as np
import jax
import jax.numpy as jnp
from jax import lax
from jax.experimental import pallas as pl
from jax.experimental.pallas import tpu as pltpu
from jax.experimental.pallas import tpu_sc as plsc

VOCAB = 100000
EMB = 16
FIELDS = 26
DENSE = 13
B = 4096
V = FIELDS * VOCAB

NC = 2
NS = 16
NW = NC * NS
BPW = B // NW
NIDX = BPW * FIELDS
NDEN = BPW * DENSE

TCH = 16384

_p = np.arange(NIDX)
_VOFF_TAB = ((_p % FIELDS) * VOCAB).astype(np.int32)
_DST_TAB = ((_p % FIELDS) * BPW + _p // FIELDS).astype(np.int32)
_q = np.arange(NDEN)
_DDST_TAB = ((_q // DENSE) * 16 + (_q % DENSE)).astype(np.int32)

_mesh = plsc.VectorSubcoreMesh(core_axis_name="c", subcore_axis_name="s")

_GATHER_DN = lax.GatherDimensionNumbers(
    offset_dims=(), collapsed_slice_dims=(0,), start_index_map=(0,)
)


def _permute(x, idx16):
    return lax.gather(
        x, idx16[:, None], _GATHER_DN, slice_sizes=(1,),
        mode=lax.GatherScatterMode.PROMISE_IN_BOUNDS,
    )


def _transpose_block(tt_ref, out_ref):
    blk = tt_ref[...]
    eye = jnp.eye(16, dtype=jnp.float32)
    out_ref[...] = lax.dot_general(
        blk, eye, (((0,), (0,)), ((), ())),
        precision=lax.Precision.HIGHEST,
        preferred_element_type=jnp.float32,
    )


_relayout = pl.pallas_call(
    _transpose_block,
    grid=(pl.cdiv(V, TCH),),
    in_specs=[pl.BlockSpec((16, TCH), lambda i: (0, i))],
    out_specs=pl.BlockSpec((TCH, 16), lambda i: (i, 0)),
    out_shape=jax.ShapeDtypeStruct((V, EMB), jnp.float32),
)


@functools.partial(
    pl.kernel,
    mesh=_mesh,
    out_type=jax.ShapeDtypeStruct((B,), jnp.float32),
    scratch_types=[
        pltpu.VMEM((NIDX,), jnp.int32),
        pltpu.VMEM((NIDX,), jnp.int32),
        pltpu.VMEM((NIDX,), jnp.int32),
        pltpu.VMEM((NIDX,), jnp.int32),
        pltpu.VMEM((NDEN,), jnp.float32),
        pltpu.VMEM((NDEN,), jnp.int32),
        pltpu.VMEM((BPW * 16,), jnp.float32),
        pltpu.VMEM((FIELDS, BPW, EMB), jnp.float32),
        pltpu.VMEM((FIELDS, EMB), jnp.float32),
        pltpu.VMEM((16,), jnp.float32),
        pltpu.VMEM((BPW,), jnp.float32),
        pltpu.SemaphoreType.DMA,
    ],
    compiler_params=pltpu.CompilerParams(
        use_tc_tiling_on_sc=False, needs_layout_passes=False
    ),
)
def _sc_forward(idx_hbm, dense_hbm, tables_hbm, voff_hbm, dst_hbm, ddst_hbm,
                wf_hbm, wd_hbm, out_hbm,
                idxf_v, idxt_v, voff_v, dst_v, denf_v, ddst_v, dblk_v,
                rows_v, wf_v, wd_v, out_v, sem):
    wid = lax.axis_index("s") * NC + lax.axis_index("c")

    pltpu.sync_copy(idx_hbm.at[pl.ds(wid * NIDX, NIDX)], idxf_v)
    pltpu.sync_copy(voff_hbm, voff_v)
    pltpu.sync_copy(dst_hbm, dst_v)

    for j in range(NIDX // 16):
        sl = pl.ds(j * 16, 16)
        plsc.store_scatter(idxt_v, [dst_v[sl]], idxf_v[sl] + voff_v[sl])

    copies = [
        pltpu.make_async_copy(
            tables_hbm.at[idxt_v.at[pl.ds(f * BPW, BPW)]], rows_v.at[f], sem
        )
        for f in range(FIELDS)
    ]
    for c in copies:
        c.start()

    pltpu.sync_copy(dense_hbm.at[pl.ds(wid * NDEN, NDEN)], denf_v)
    pltpu.sync_copy(ddst_hbm, ddst_v)
    pltpu.sync_copy(wf_hbm, wf_v)
    pltpu.sync_copy(wd_hbm, wd_v)
    zero16 = jnp.zeros((16,), jnp.float32)
    for j in range(BPW):
        dblk_v[pl.ds(j * 16, 16)] = zero16
    for j in range(NDEN // 16):
        sl = pl.ds(j * 16, 16)
        plsc.store_scatter(dblk_v, [ddst_v[sl]], denf_v[sl])

    for c in copies:
        c.wait()

    wfs = [wf_v[f] for f in range(FIELDS)]
    wdv = wd_v[...]
    lane = lax.iota(jnp.int32, 16)
    perms = [lane ^ sh for sh in (8, 4, 2, 1)]

    for g in range(BPW // 16):
        def row_body(b, out16):
            i = g * 16 + b
            acc = dblk_v[pl.ds(i * 16, 16)] * wdv
            for f in range(FIELDS):
                acc = acc + rows_v[f, i] * wfs[f]
            for p in perms:
                acc = acc + _permute(acc, p)
            return jnp.where(lane == b, acc, out16)

        out16 = lax.fori_loop(0, 16, row_body, jnp.zeros((16,), jnp.float32))
        out_v[pl.ds(g * 16, 16)] = 1.0 / (1.0 + jnp.exp(-out16))

    pltpu.sync_copy(out_v, out_hbm.at[pl.ds(wid * BPW, BPW)])


@jax.jit
def kernel(sparse_idx, dense, tables, W):
    idx_flat = sparse_idx.astype(jnp.int32).reshape(-1)
    dense_flat = dense.reshape(-1)
    wf = W[: FIELDS * EMB, 0].reshape(FIELDS, EMB)
    wd = jnp.concatenate([W[FIELDS * EMB :, 0], jnp.zeros((16 - DENSE,), jnp.float32)])
    tables_rm = _relayout(tables.T)
    out = _sc_forward(
        idx_flat, dense_flat, tables_rm,
        jnp.asarray(_VOFF_TAB), jnp.asarray(_DST_TAB), jnp.asarray(_DDST_TAB),
        wf, wd,
    )
    return out.reshape(B, 1)


# speedup vs baseline: 1.4752x; 1.4752x over previous
import functools

import numpy as np
import jax
import jax.numpy as jnp
from jax import lax
from jax.experimental import pallas as pl
from jax.experimental.pallas import tpu as pltpu
from jax.experimental.pallas import tpu_sc as plsc

VOCAB = 100000
EMB = 16
FIELDS = 26
DENSE = 13
B = 4096
V = FIELDS * VOCAB

NC = 2
NS = 16
NW = NC * NS
BPW = B // NW
NIDX = BPW * FIELDS
NDEN = BPW * DENSE

TCH = 16384

_p = np.arange(NIDX)
_VOFF_TAB = ((_p % FIELDS) * VOCAB).astype(np.int32)
_DST_TAB = ((_p % FIELDS) * BPW + _p // FIELDS).astype(np.int32)
_q = np.arange(NDEN)
_DDST_TAB = ((_q // DENSE) * 16 + (_q % DENSE)).astype(np.int32)

_mesh = plsc.VectorSubcoreMesh(core_axis_name="c", subcore_axis_name="s")

_GATHER_DN = lax.GatherDimensionNumbers(
    offset_dims=(), collapsed_slice_dims=(0,), start_index_map=(0,)
)


def _permute(x, idx16):
    return lax.gather(
        x, idx16[:, None], _GATHER_DN, slice_sizes=(1,),
        mode=lax.GatherScatterMode.PROMISE_IN_BOUNDS,
    )


def _transpose_block(tt_ref, out_ref):
    out_ref[...] = tt_ref[...].T


_relayout = pl.pallas_call(
    _transpose_block,
    grid=(pl.cdiv(V, TCH),),
    in_specs=[pl.BlockSpec((16, TCH), lambda i: (0, i))],
    out_specs=pl.BlockSpec((TCH, 16), lambda i: (i, 0)),
    out_shape=jax.ShapeDtypeStruct((V, EMB), jnp.float32),
)


@functools.partial(
    pl.kernel,
    mesh=_mesh,
    out_type=jax.ShapeDtypeStruct((B,), jnp.float32),
    scratch_types=[
        pltpu.VMEM((NIDX,), jnp.int32),
        pltpu.VMEM((NIDX,), jnp.int32),
        pltpu.VMEM((NIDX,), jnp.int32),
        pltpu.VMEM((NIDX,), jnp.int32),
        pltpu.VMEM((NDEN,), jnp.float32),
        pltpu.VMEM((NDEN,), jnp.int32),
        pltpu.VMEM((BPW * 16,), jnp.float32),
        pltpu.VMEM((FIELDS, BPW, EMB), jnp.float32),
        pltpu.VMEM((FIELDS, EMB), jnp.float32),
        pltpu.VMEM((16,), jnp.float32),
        pltpu.VMEM((BPW,), jnp.float32),
        pltpu.SemaphoreType.DMA,
    ],
    compiler_params=pltpu.CompilerParams(
        use_tc_tiling_on_sc=False, needs_layout_passes=False
    ),
)
def _sc_forward(idx_hbm, dense_hbm, tables_hbm, voff_hbm, dst_hbm, ddst_hbm,
                wf_hbm, wd_hbm, out_hbm,
                idxf_v, idxt_v, voff_v, dst_v, denf_v, ddst_v, dblk_v,
                rows_v, wf_v, wd_v, out_v, sem):
    wid = lax.axis_index("s") * NC + lax.axis_index("c")

    pltpu.sync_copy(idx_hbm.at[pl.ds(wid * NIDX, NIDX)], idxf_v)
    pltpu.sync_copy(voff_hbm, voff_v)
    pltpu.sync_copy(dst_hbm, dst_v)

    for j in range(NIDX // 16):
        sl = pl.ds(j * 16, 16)
        plsc.store_scatter(idxt_v, [dst_v[sl]], idxf_v[sl] + voff_v[sl])

    copies = [
        pltpu.make_async_copy(
            tables_hbm.at[idxt_v.at[pl.ds(f * BPW, BPW)]], rows_v.at[f], sem
        )
        for f in range(FIELDS)
    ]
    for c in copies:
        c.start()

    pltpu.sync_copy(dense_hbm.at[pl.ds(wid * NDEN, NDEN)], denf_v)
    pltpu.sync_copy(ddst_hbm, ddst_v)
    pltpu.sync_copy(wf_hbm, wf_v)
    pltpu.sync_copy(wd_hbm, wd_v)
    zero16 = jnp.zeros((16,), jnp.float32)
    for j in range(BPW):
        dblk_v[pl.ds(j * 16, 16)] = zero16
    for j in range(NDEN // 16):
        sl = pl.ds(j * 16, 16)
        plsc.store_scatter(dblk_v, [ddst_v[sl]], denf_v[sl])

    for c in copies:
        c.wait()

    wfs = [wf_v[f] for f in range(FIELDS)]
    wdv = wd_v[...]
    lane = lax.iota(jnp.int32, 16)
    perms = [lane ^ sh for sh in (8, 4, 2, 1)]

    for g in range(BPW // 16):
        def row_body(b, out16):
            i = g * 16 + b
            acc = dblk_v[pl.ds(i * 16, 16)] * wdv
            for f in range(FIELDS):
                acc = acc + rows_v[f, i] * wfs[f]
            for p in perms:
                acc = acc + _permute(acc, p)
            return jnp.where(lane == b, acc, out16)

        out16 = lax.fori_loop(0, 16, row_body, jnp.zeros((16,), jnp.float32))
        out_v[pl.ds(g * 16, 16)] = 1.0 / (1.0 + jnp.exp(-out16))

    pltpu.sync_copy(out_v, out_hbm.at[pl.ds(wid * BPW, BPW)])


@jax.jit
def kernel(sparse_idx, dense, tables, W):
    idx_flat = sparse_idx.astype(jnp.int32).reshape(-1)
    dense_flat = dense.reshape(-1)
    wf = W[: FIELDS * EMB, 0].reshape(FIELDS, EMB)
    wd = jnp.concatenate([W[FIELDS * EMB :, 0], jnp.zeros((16 - DENSE,), jnp.float32)])
    tables_rm = _relayout(tables.T)
    out = _sc_forward(
        idx_flat, dense_flat, tables_rm,
        jnp.asarray(_VOFF_TAB), jnp.asarray(_DST_TAB), jnp.asarray(_DDST_TAB),
        wf, wd,
    )
    return out.reshape(B, 1)


# speedup vs baseline: 8.7533x; 5.9338x over previous
import functools

import numpy as np
import jax
import jax.numpy as jnp
from jax import lax
from jax.experimental import pallas as pl
from jax.experimental.pallas import tpu as pltpu
from jax.experimental.pallas import tpu_sc as plsc

VOCAB = 100000
EMB = 16
FIELDS = 26
DENSE = 13
B = 4096
V = FIELDS * VOCAB

NC = 2
NS = 16
NW = NC * NS
BPW = B // NW
NIDX = BPW * FIELDS
NDEN = BPW * DENSE

TCH = 16384
TSUB = TCH // 8
NSTEP = (V + TCH - 1) // TCH
VPAD = NSTEP * TCH

_p = np.arange(NIDX)
_VOFF_TAB = ((_p % FIELDS) * VOCAB).astype(np.int32)
_DST_TAB = ((_p % FIELDS) * BPW + _p // FIELDS).astype(np.int32)
_q = np.arange(NDEN)
_DDST_TAB = ((_q // DENSE) * 16 + (_q % DENSE)).astype(np.int32)

_mesh = plsc.VectorSubcoreMesh(core_axis_name="c", subcore_axis_name="s")

_GATHER_DN = lax.GatherDimensionNumbers(
    offset_dims=(), collapsed_slice_dims=(0,), start_index_map=(0,)
)


def _permute(x, idx16):
    return lax.gather(
        x, idx16[:, None], _GATHER_DN, slice_sizes=(1,),
        mode=lax.GatherScatterMode.PROMISE_IN_BOUNDS,
    )


def _transpose_block(tt_ref, out_ref):
    blk = tt_ref[...]
    x = jnp.concatenate(
        [blk[:, k * TSUB:(k + 1) * TSUB] for k in range(8)], axis=0
    )
    out_ref[...] = x.T


_relayout = pl.pallas_call(
    _transpose_block,
    grid=(NSTEP,),
    in_specs=[pl.BlockSpec((16, TCH), lambda i: (0, i))],
    out_specs=pl.BlockSpec((TSUB, 128), lambda i: (i, 0)),
    out_shape=jax.ShapeDtypeStruct((VPAD // 8, 128), jnp.float32),
)


@functools.partial(
    pl.kernel,
    mesh=_mesh,
    out_type=jax.ShapeDtypeStruct((B,), jnp.float32),
    scratch_types=[
        pltpu.VMEM((NIDX,), jnp.int32),
        pltpu.VMEM((NIDX,), jnp.int32),
        pltpu.VMEM((NIDX,), jnp.int32),
        pltpu.VMEM((NIDX,), jnp.int32),
        pltpu.VMEM((NDEN,), jnp.float32),
        pltpu.VMEM((NDEN,), jnp.int32),
        pltpu.VMEM((BPW * 16,), jnp.float32),
        pltpu.VMEM((FIELDS, BPW, EMB), jnp.float32),
        pltpu.VMEM((FIELDS, EMB), jnp.float32),
        pltpu.VMEM((16,), jnp.float32),
        pltpu.VMEM((BPW,), jnp.float32),
        pltpu.SemaphoreType.DMA,
    ],
    compiler_params=pltpu.CompilerParams(
        use_tc_tiling_on_sc=False, needs_layout_passes=False
    ),
)
def _sc_forward(idx_hbm, dense_hbm, tables_hbm, voff_hbm, dst_hbm, ddst_hbm,
                wf_hbm, wd_hbm, out_hbm,
                idxf_v, idxt_v, voff_v, dst_v, denf_v, ddst_v, dblk_v,
                rows_v, wf_v, wd_v, out_v, sem):
    wid = lax.axis_index("s") * NC + lax.axis_index("c")

    pltpu.sync_copy(idx_hbm.at[pl.ds(wid * NIDX, NIDX)], idxf_v)
    pltpu.sync_copy(voff_hbm, voff_v)
    pltpu.sync_copy(dst_hbm, dst_v)

    for j in range(NIDX // 16):
        sl = pl.ds(j * 16, 16)
        va = idxf_v[sl] + voff_v[sl]
        p = (
            (va & jnp.int32(~16383))
            + lax.shift_left(va & jnp.int32(2047), jnp.int32(3))
            + (lax.shift_right_logical(va, jnp.int32(11)) & jnp.int32(7))
        )
        plsc.store_scatter(idxt_v, [dst_v[sl]], p)

    copies = [
        pltpu.make_async_copy(
            tables_hbm.at[idxt_v.at[pl.ds(f * BPW, BPW)]], rows_v.at[f], sem
        )
        for f in range(FIELDS)
    ]
    for c in copies:
        c.start()

    pltpu.sync_copy(dense_hbm.at[pl.ds(wid * NDEN, NDEN)], denf_v)
    pltpu.sync_copy(ddst_hbm, ddst_v)
    pltpu.sync_copy(wf_hbm, wf_v)
    pltpu.sync_copy(wd_hbm, wd_v)
    zero16 = jnp.zeros((16,), jnp.float32)
    for j in range(BPW):
        dblk_v[pl.ds(j * 16, 16)] = zero16
    for j in range(NDEN // 16):
        sl = pl.ds(j * 16, 16)
        plsc.store_scatter(dblk_v, [ddst_v[sl]], denf_v[sl])

    for c in copies:
        c.wait()

    wfs = [wf_v[f] for f in range(FIELDS)]
    wdv = wd_v[...]
    lane = lax.iota(jnp.int32, 16)
    perms = [lane ^ sh for sh in (8, 4, 2, 1)]

    for g in range(BPW // 16):
        def row_body(b, out16):
            i = g * 16 + b
            acc = dblk_v[pl.ds(i * 16, 16)] * wdv
            for f in range(FIELDS):
                acc = acc + rows_v[f, i] * wfs[f]
            for p in perms:
                acc = acc + _permute(acc, p)
            return jnp.where(lane == b, acc, out16)

        out16 = lax.fori_loop(0, 16, row_body, jnp.zeros((16,), jnp.float32))
        out_v[pl.ds(g * 16, 16)] = 1.0 / (1.0 + jnp.exp(-out16))

    pltpu.sync_copy(out_v, out_hbm.at[pl.ds(wid * BPW, BPW)])


@jax.jit
def kernel(sparse_idx, dense, tables, W):
    idx_flat = sparse_idx.astype(jnp.int32).reshape(-1)
    dense_flat = dense.reshape(-1)
    wf = W[: FIELDS * EMB, 0].reshape(FIELDS, EMB)
    wd = jnp.concatenate([W[FIELDS * EMB :, 0], jnp.zeros((16 - DENSE,), jnp.float32)])
    tables_rm = _relayout(tables.T).reshape(VPAD, EMB)
    out = _sc_forward(
        idx_flat, dense_flat, tables_rm,
        jnp.asarray(_VOFF_TAB), jnp.asarray(_DST_TAB), jnp.asarray(_DDST_TAB),
        wf, wd,
    )
    return out.reshape(B, 1)


# speedup vs baseline: 12.4659x; 1.4241x over previous
import functools

import numpy as np
import jax
import jax.numpy as jnp
from jax import lax
from jax.experimental import pallas as pl
from jax.experimental.pallas import tpu as pltpu
from jax.experimental.pallas import tpu_sc as plsc

VOCAB = 100000
EMB = 16
FIELDS = 26
DENSE = 13
B = 4096
V = FIELDS * VOCAB

NC = 2
NS = 16
NW = NC * NS
BPW = B // NW
NIDX = BPW * FIELDS
NDEN = BPW * DENSE

TCH = 65536
TSUB = TCH // 8
TSUB_BITS = TSUB.bit_length() - 1
NSTEP = (V + TCH - 1) // TCH
VPAD = NSTEP * TCH

_p = np.arange(NIDX)
_VOFF_TAB = ((_p % FIELDS) * VOCAB).astype(np.int32)
_DST_TAB = ((_p % FIELDS) * BPW + _p // FIELDS).astype(np.int32)
_q = np.arange(NDEN)
_DDST_TAB = ((_q // DENSE) * 16 + (_q % DENSE)).astype(np.int32)

_mesh = plsc.VectorSubcoreMesh(core_axis_name="c", subcore_axis_name="s")

_GATHER_DN = lax.GatherDimensionNumbers(
    offset_dims=(), collapsed_slice_dims=(0,), start_index_map=(0,)
)


def _permute(x, idx16):
    return lax.gather(
        x, idx16[:, None], _GATHER_DN, slice_sizes=(1,),
        mode=lax.GatherScatterMode.PROMISE_IN_BOUNDS,
    )


def _transpose_block(tt_ref, out_ref):
    blk = tt_ref[...]
    x = jnp.concatenate(
        [blk[:, k * TSUB:(k + 1) * TSUB] for k in range(8)], axis=0
    )
    out_ref[...] = x.T


_relayout = pl.pallas_call(
    _transpose_block,
    grid=(NSTEP,),
    in_specs=[pl.BlockSpec((16, TCH), lambda i: (0, i))],
    out_specs=pl.BlockSpec((TSUB, 128), lambda i: (i, 0)),
    out_shape=jax.ShapeDtypeStruct((VPAD // 8, 128), jnp.float32),
)


@functools.partial(
    pl.kernel,
    mesh=_mesh,
    out_type=jax.ShapeDtypeStruct((B,), jnp.float32),
    scratch_types=[
        pltpu.VMEM((NIDX,), jnp.int32),
        pltpu.VMEM((NIDX,), jnp.int32),
        pltpu.VMEM((NIDX,), jnp.int32),
        pltpu.VMEM((NIDX,), jnp.int32),
        pltpu.VMEM((NDEN,), jnp.float32),
        pltpu.VMEM((NDEN,), jnp.int32),
        pltpu.VMEM((BPW * 16,), jnp.float32),
        pltpu.VMEM((FIELDS, BPW, EMB), jnp.float32),
        pltpu.VMEM((FIELDS, EMB), jnp.float32),
        pltpu.VMEM((16,), jnp.float32),
        pltpu.VMEM((BPW,), jnp.float32),
        pltpu.SemaphoreType.DMA,
    ],
    compiler_params=pltpu.CompilerParams(
        use_tc_tiling_on_sc=False, needs_layout_passes=False
    ),
)
def _sc_forward(idx_hbm, dense_hbm, tables_hbm, voff_hbm, dst_hbm, ddst_hbm,
                wf_hbm, wd_hbm, out_hbm,
                idxf_v, idxt_v, voff_v, dst_v, denf_v, ddst_v, dblk_v,
                rows_v, wf_v, wd_v, out_v, sem):
    wid = lax.axis_index("s") * NC + lax.axis_index("c")

    pltpu.sync_copy(idx_hbm.at[pl.ds(wid * NIDX, NIDX)], idxf_v)
    pltpu.sync_copy(voff_hbm, voff_v)
    pltpu.sync_copy(dst_hbm, dst_v)

    for j in range(NIDX // 16):
        sl = pl.ds(j * 16, 16)
        va = idxf_v[sl] + voff_v[sl]
        p = (
            (va & jnp.int32(~(TCH - 1)))
            + lax.shift_left(va & jnp.int32(TSUB - 1), jnp.int32(3))
            + (lax.shift_right_logical(va, jnp.int32(TSUB_BITS)) & jnp.int32(7))
        )
        plsc.store_scatter(idxt_v, [dst_v[sl]], p)

    copies = [
        pltpu.make_async_copy(
            tables_hbm.at[idxt_v.at[pl.ds(f * BPW, BPW)]], rows_v.at[f], sem
        )
        for f in range(FIELDS)
    ]
    for c in copies:
        c.start()

    pltpu.sync_copy(dense_hbm.at[pl.ds(wid * NDEN, NDEN)], denf_v)
    pltpu.sync_copy(ddst_hbm, ddst_v)
    pltpu.sync_copy(wf_hbm, wf_v)
    pltpu.sync_copy(wd_hbm, wd_v)
    zero16 = jnp.zeros((16,), jnp.float32)
    for j in range(BPW):
        dblk_v[pl.ds(j * 16, 16)] = zero16
    for j in range(NDEN // 16):
        sl = pl.ds(j * 16, 16)
        plsc.store_scatter(dblk_v, [ddst_v[sl]], denf_v[sl])

    for c in copies:
        c.wait()

    wfs = [wf_v[f] for f in range(FIELDS)]
    wdv = wd_v[...]
    lane = lax.iota(jnp.int32, 16)
    perms = [lane ^ sh for sh in (8, 4, 2, 1)]

    for g in range(BPW // 16):
        def row_body(b, out16):
            i = g * 16 + b
            acc = dblk_v[pl.ds(i * 16, 16)] * wdv
            for f in range(FIELDS):
                acc = acc + rows_v[f, i] * wfs[f]
            for p in perms:
                acc = acc + _permute(acc, p)
            return jnp.where(lane == b, acc, out16)

        out16 = lax.fori_loop(0, 16, row_body, jnp.zeros((16,), jnp.float32))
        out_v[pl.ds(g * 16, 16)] = 1.0 / (1.0 + jnp.exp(-out16))

    pltpu.sync_copy(out_v, out_hbm.at[pl.ds(wid * BPW, BPW)])


@jax.jit
def kernel(sparse_idx, dense, tables, W):
    idx_flat = sparse_idx.astype(jnp.int32).reshape(-1)
    dense_flat = dense.reshape(-1)
    wf = W[: FIELDS * EMB, 0].reshape(FIELDS, EMB)
    wd = jnp.concatenate([W[FIELDS * EMB :, 0], jnp.zeros((16 - DENSE,), jnp.float32)])
    tables_rm = _relayout(tables.T).reshape(VPAD, EMB)
    out = _sc_forward(
        idx_flat, dense_flat, tables_rm,
        jnp.asarray(_VOFF_TAB), jnp.asarray(_DST_TAB), jnp.asarray(_DDST_TAB),
        wf, wd,
    )
    return out.reshape(B, 1)


# speedup vs baseline: 12.6720x; 1.0165x over previous
import functools

import numpy as np
import jax
import jax.numpy as jnp
from jax import lax
from jax.experimental import pallas as pl
from jax.experimental.pallas import tpu as pltpu
from jax.experimental.pallas import tpu_sc as plsc

VOCAB = 100000
EMB = 16
FIELDS = 26
DENSE = 13
B = 4096
V = FIELDS * VOCAB

NC = 2
NS = 16
NW = NC * NS
BPW = B // NW
NIDX = BPW * FIELDS
NDEN = BPW * DENSE

TCH = 131072
TSUB = TCH // 8
TSUB_BITS = TSUB.bit_length() - 1
NSTEP = (V + TCH - 1) // TCH
VPAD = NSTEP * TCH

_p = np.arange(NIDX)
_VOFF_TAB = ((_p % FIELDS) * VOCAB).astype(np.int32)
_DST_TAB = ((_p % FIELDS) * BPW + _p // FIELDS).astype(np.int32)
_q = np.arange(NDEN)
_DDST_TAB = ((_q // DENSE) * 16 + (_q % DENSE)).astype(np.int32)

_mesh = plsc.VectorSubcoreMesh(core_axis_name="c", subcore_axis_name="s")

_GATHER_DN = lax.GatherDimensionNumbers(
    offset_dims=(), collapsed_slice_dims=(0,), start_index_map=(0,)
)


def _permute(x, idx16):
    return lax.gather(
        x, idx16[:, None], _GATHER_DN, slice_sizes=(1,),
        mode=lax.GatherScatterMode.PROMISE_IN_BOUNDS,
    )


def _transpose_block(tt_ref, out_ref):
    blk = tt_ref[...]
    x = jnp.concatenate(
        [blk[:, k * TSUB:(k + 1) * TSUB] for k in range(8)], axis=0
    )
    out_ref[...] = x.T


_relayout = pl.pallas_call(
    _transpose_block,
    grid=(NSTEP,),
    in_specs=[pl.BlockSpec((16, TCH), lambda i: (0, i))],
    out_specs=pl.BlockSpec((TSUB, 128), lambda i: (i, 0)),
    out_shape=jax.ShapeDtypeStruct((VPAD // 8, 128), jnp.float32),
)


@functools.partial(
    pl.kernel,
    mesh=_mesh,
    out_type=jax.ShapeDtypeStruct((B,), jnp.float32),
    scratch_types=[
        pltpu.VMEM((NIDX,), jnp.int32),
        pltpu.VMEM((NIDX,), jnp.int32),
        pltpu.VMEM((NIDX,), jnp.int32),
        pltpu.VMEM((NIDX,), jnp.int32),
        pltpu.VMEM((NDEN,), jnp.float32),
        pltpu.VMEM((NDEN,), jnp.int32),
        pltpu.VMEM((BPW * 16,), jnp.float32),
        pltpu.VMEM((FIELDS, BPW, EMB), jnp.float32),
        pltpu.VMEM((FIELDS, EMB), jnp.float32),
        pltpu.VMEM((16,), jnp.float32),
        pltpu.VMEM((BPW,), jnp.float32),
        pltpu.SemaphoreType.DMA,
    ],
    compiler_params=pltpu.CompilerParams(
        use_tc_tiling_on_sc=False, needs_layout_passes=False
    ),
)
def _sc_forward(idx_hbm, dense_hbm, tables_hbm, voff_hbm, dst_hbm, ddst_hbm,
                wf_hbm, wd_hbm, out_hbm,
                idxf_v, idxt_v, voff_v, dst_v, denf_v, ddst_v, dblk_v,
                rows_v, wf_v, wd_v, out_v, sem):
    wid = lax.axis_index("s") * NC + lax.axis_index("c")

    pltpu.sync_copy(idx_hbm.at[pl.ds(wid * NIDX, NIDX)], idxf_v)
    pltpu.sync_copy(voff_hbm, voff_v)
    pltpu.sync_copy(dst_hbm, dst_v)

    for j in range(NIDX // 16):
        sl = pl.ds(j * 16, 16)
        va = idxf_v[sl] + voff_v[sl]
        p = (
            (va & jnp.int32(~(TCH - 1)))
            + lax.shift_left(va & jnp.int32(TSUB - 1), jnp.int32(3))
            + (lax.shift_right_logical(va, jnp.int32(TSUB_BITS)) & jnp.int32(7))
        )
        plsc.store_scatter(idxt_v, [dst_v[sl]], p)

    copies = [
        pltpu.make_async_copy(
            tables_hbm.at[idxt_v.at[pl.ds(f * BPW, BPW)]], rows_v.at[f], sem
        )
        for f in range(FIELDS)
    ]
    for c in copies:
        c.start()

    pltpu.sync_copy(dense_hbm.at[pl.ds(wid * NDEN, NDEN)], denf_v)
    pltpu.sync_copy(ddst_hbm, ddst_v)
    pltpu.sync_copy(wf_hbm, wf_v)
    pltpu.sync_copy(wd_hbm, wd_v)
    zero16 = jnp.zeros((16,), jnp.float32)
    for j in range(BPW):
        dblk_v[pl.ds(j * 16, 16)] = zero16
    for j in range(NDEN // 16):
        sl = pl.ds(j * 16, 16)
        plsc.store_scatter(dblk_v, [ddst_v[sl]], denf_v[sl])

    for c in copies:
        c.wait()

    wfs = [wf_v[f] for f in range(FIELDS)]
    wdv = wd_v[...]
    lane = lax.iota(jnp.int32, 16)
    perms = [lane ^ sh for sh in (8, 4, 2, 1)]

    for g in range(BPW // 16):
        def row_body(b, out16):
            i = g * 16 + b
            acc = dblk_v[pl.ds(i * 16, 16)] * wdv
            for f in range(FIELDS):
                acc = acc + rows_v[f, i] * wfs[f]
            for p in perms:
                acc = acc + _permute(acc, p)
            return jnp.where(lane == b, acc, out16)

        out16 = lax.fori_loop(0, 16, row_body, jnp.zeros((16,), jnp.float32))
        out_v[pl.ds(g * 16, 16)] = 1.0 / (1.0 + jnp.exp(-out16))

    pltpu.sync_copy(out_v, out_hbm.at[pl.ds(wid * BPW, BPW)])


@jax.jit
def kernel(sparse_idx, dense, tables, W):
    idx_flat = sparse_idx.astype(jnp.int32).reshape(-1)
    dense_flat = dense.reshape(-1)
    wf = W[: FIELDS * EMB, 0].reshape(FIELDS, EMB)
    wd = jnp.concatenate([W[FIELDS * EMB :, 0], jnp.zeros((16 - DENSE,), jnp.float32)])
    tables_rm = _relayout(tables.T).reshape(VPAD, EMB)
    out = _sc_forward(
        idx_flat, dense_flat, tables_rm,
        jnp.asarray(_VOFF_TAB), jnp.asarray(_DST_TAB), jnp.asarray(_DDST_TAB),
        wf, wd,
    )
    return out.reshape(B, 1)
